# baseline (speedup 1.0000x reference)
import jax
import jax.numpy as jnp
from jax import lax
from jax.experimental import pallas as pl
from jax.experimental.pallas import tpu as pltpu
from jax.experimental.pallas import tpu_sc as plsc

NC, NS = 2, 16
CHUNK = 32
GROUP = 128
SUPER = 8
PAD_ROWS = 64
EPS = 1e-5
BN = 1000


def _mesh():
    return plsc.VectorSubcoreMesh(core_axis_name="c", subcore_axis_name="s",
                                  num_cores=NC, num_subcores=NS)


def _make_agg(n_chunks, n_acc, n_out, e_pad, with_count=False):
    per_core = n_chunks // NC
    g_total = e_pad // GROUP
    g_tile = g_total // NS
    nsb = g_tile // SUPER
    z_sl = n_out // NS

    def chunk_pass(j, tbl, out, src2, dst2, zeros, acc, si, s4, di, rb,
                   gsem, ssem, isem, s, cnt_refs):
        pltpu.sync_copy(zeros, acc.at[pl.ds(s * z_sl, z_sl)])
        if cnt_refs is not None:
            acc_cnt, ones, zeros1, cnt_out = cnt_refs
            for k in range(GROUP // 16):
                ones[pl.ds(k * 16, 16)] = jnp.ones((16,), jnp.float32)
            pltpu.sync_copy(zeros1, acc_cnt.at[pl.ds(s * z_sl, z_sl)])
        plsc.subcore_barrier()
        g0 = s * g_tile

        def load_idx(sbk, sl):
            g = g0 + sbk * SUPER
            pltpu.async_copy(src2.at[pl.ds(g, SUPER)], si[sl], isem)
            pltpu.async_copy(dst2.at[pl.ds(g, SUPER)], di[sl], isem)

        def wait_idx(sbk, sl):
            g = g0 + sbk * SUPER
            pltpu.make_async_copy(src2.at[pl.ds(g, SUPER)], si[sl],
                                  isem).wait()
            pltpu.make_async_copy(dst2.at[pl.ds(g, SUPER)], di[sl],
                                  isem).wait()

        def calc_s4(sl):
            for gi in range(SUPER):
                for k in range(GROUP // 16):
                    s4[sl][gi, pl.ds(k * 16, 16)] = (
                        si[sl][gi, pl.ds(k * 16, 16)] * n_chunks + j)

        def gissue(sbk, gi, isl, rsl):
            pltpu.async_copy(tbl.at[s4[isl].at[gi]], rb[rsl], gsem[rsl])

        def gwait(isl, gi, rsl):
            pltpu.make_async_copy(tbl.at[s4[isl].at[gi]], rb[rsl],
                                  gsem[rsl]).wait()

        def sissue(isl, gi, rsl):
            pltpu.async_copy(rb[rsl], acc.at[di[isl].at[gi]],
                             ssem[rsl], add=True)
            if cnt_refs is not None:
                pltpu.async_copy(ones, acc_cnt.at[di[isl].at[gi]],
                                 ssem[rsl], add=True)

        def swait(rsl):
            pltpu.make_async_copy(rb[rsl], acc.at[pl.ds(0, GROUP)],
                                  ssem[rsl]).wait()
            if cnt_refs is not None:
                pltpu.make_async_copy(ones, acc_cnt.at[pl.ds(0, GROUP)],
                                      ssem[rsl]).wait()

        load_idx(0, 0)
        wait_idx(0, 0)
        calc_s4(0)
        for gg in range(2):
            gissue(0, gg, 0, gg)

        def sblock(sbk, carry):
            isl_d = lax.rem(sbk, 2)

            def do(cur_par):
                isl = cur_par
                nxt = 1 - cur_par
                for gi in range(SUPER):
                    rsl = gi % 4
                    gwait(isl, gi, rsl)
                    sissue(isl, gi, rsl)
                    nsl = (gi + 2) % 4
                    if gi == 0:
                        pl.when(sbk < nsb - 1)(
                            lambda: load_idx(sbk + 1, nxt))
                    if gi < 2:
                        pl.when(sbk > 0)(lambda nsl=nsl: swait(nsl))
                    else:
                        swait(nsl)
                    if gi == 4:
                        def prep():
                            wait_idx(sbk + 1, nxt)
                            calc_s4(nxt)
                        pl.when(sbk < nsb - 1)(prep)
                    if gi < 6:
                        gissue(sbk, gi + 2, isl, nsl)
                    else:
                        pl.when(sbk < nsb - 1)(
                            lambda gi=gi, nsl=nsl:
                            gissue(sbk + 1, gi - 6, nxt, nsl))

            for par in range(2):
                pl.when(isl_d == par)(lambda par=par: do(par))
            return carry

        lax.fori_loop(0, nsb, sblock, 0)
        swait((g_tile - 2) % 4)
        swait((g_tile - 1) % 4)
        plsc.subcore_barrier()
        pltpu.sync_copy(acc.at[pl.ds(s * z_sl, z_sl)],
                        out.at[pl.ds(s * z_sl, z_sl),
                               pl.ds(j * CHUNK, CHUNK)])
        if cnt_refs is not None:
            pltpu.sync_copy(acc_cnt.at[pl.ds(s * z_sl, z_sl)],
                            cnt_out.at[pl.ds(s * z_sl, z_sl)])
        plsc.subcore_barrier()

    n_in = 5 if with_count else 4

    def body(*refs):
        tbl = refs[0]
        src2, dst2, zeros = refs[1:4]
        zeros1 = refs[4] if with_count else None
        out = refs[n_in]
        cnt_out = refs[n_in + 1] if with_count else None
        sc = refs[n_in + (2 if with_count else 1):]
        acc = sc[0]
        si, s4, di = sc[1:3], sc[3:5], sc[5:7]
        rb = sc[7:11]
        gsem, ssem = sc[11:15], sc[15:19]
        isem = sc[19]
        acc_cnt = sc[20] if with_count else None
        ones = sc[21] if with_count else None
        c = lax.axis_index("c")
        s = lax.axis_index("s")
        for cc in range(NC):
            def run(cc=cc):
                for p in range(per_core):
                    j = cc * per_core + p
                    cr = None
                    if with_count and cc == 0 and p == 0:
                        cr = (acc_cnt, ones, zeros1, cnt_out)
                    chunk_pass(j, tbl, out, src2, dst2, zeros,
                               acc, si, s4, di, rb, gsem, ssem, isem, s, cr)
            pl.when(c == cc)(run)

    out_type = (jax.ShapeDtypeStruct((n_out, n_chunks * CHUNK),
                                     jnp.float32),)
    if with_count:
        out_type = out_type + (jax.ShapeDtypeStruct((n_out,), jnp.float32),)
    scratch = (
        [pltpu.VMEM_SHARED((n_acc, CHUNK), jnp.float32)]
        + [pltpu.VMEM((SUPER, GROUP), jnp.int32) for _ in range(6)]
        + [pltpu.VMEM((GROUP, CHUNK), jnp.float32) for _ in range(4)]
        + [pltpu.SemaphoreType.DMA for _ in range(9)]
    )
    if with_count:
        scratch += [
            pltpu.VMEM_SHARED((n_acc,), jnp.float32),
            pltpu.VMEM((GROUP,), jnp.float32),
        ]
    return pl.kernel(body, out_type=out_type, mesh=_mesh(),
                     scratch_types=scratch,
                     compiler_params=pltpu.CompilerParams(
                         use_tc_tiling_on_sc=False))


def _layer_call(agg, h, cnt, wl, wr, bl, g, bb, n, head=None):
    nb = n // BN
    d = wl.shape[0]

    def common_z(agg_r, h_r, cnt_r, wl_r, wr_r, bl_r, z_scr, st_scr, i):
        @pl.when(i == 0)
        def _():
            st_scr[...] = jnp.zeros_like(st_scr)
        inv = 1.0 / jnp.maximum(cnt_r[...][:, 0], 1.0)
        z = (jnp.dot(agg_r[...], wl_r[...],
                     preferred_element_type=jnp.float32) * inv[:, None]
             + jnp.dot(h_r[...], wr_r[...],
                       preferred_element_type=jnp.float32)
             + bl_r[...])
        z_scr[pl.ds(i * BN, BN), :] = z
        st_scr[...] += jnp.stack([jnp.sum(z, axis=0),
                                  jnp.sum(z * z, axis=0)])

    def norm(z_scr, st_scr, g_r, b_r, i):
        stats = st_scr[...]
        mu = stats[0] * (1.0 / n)
        var = stats[1] * (1.0 / n) - mu * mu
        z = z_scr[pl.ds(i * BN, BN), :]
        return jnp.maximum(
            g_r[...] * (z - mu[None, :]) / jnp.sqrt(var + EPS)[None, :]
            + b_r[...], 0.0)

    if head is None:
        def kern(agg_r, h_r, cnt_r, wl_r, wr_r, bl_r, g_r, b_r,
                 h_out, z_scr, st_scr):
            p, i = pl.program_id(0), pl.program_id(1)
            pl.when(p == 0)(lambda: common_z(agg_r, h_r, cnt_r, wl_r,
                                             wr_r, bl_r, z_scr, st_scr, i))

            @pl.when(p == 1)
            def _():
                h_out[...] = norm(z_scr, st_scr, g_r, b_r, i)

        extra_in = []
        out_spec = pl.BlockSpec((BN, 128), lambda p, i: (p * i, 0))
        out_shape = jax.ShapeDtypeStruct((n, 128), jnp.float32)
        args = ()
    else:
        w1, b1, w2, b2 = head
        hd = w1.shape[1]

        def kern(agg_r, h_r, cnt_r, wl_r, wr_r, bl_r, g_r, b_r,
                 w1_r, b1_r, w2_r, b2_r, o_out, z_scr, st_scr):
            p, i = pl.program_id(0), pl.program_id(1)
            pl.when(p == 0)(lambda: common_z(agg_r, h_r, cnt_r, wl_r,
                                             wr_r, bl_r, z_scr, st_scr, i))

            @pl.when(p == 1)
            def _():
                hh = norm(z_scr, st_scr, g_r, b_r, i)
                h1 = jnp.maximum(
                    jnp.dot(hh, w1_r[...],
                            preferred_element_type=jnp.float32)
                    + b1_r[...], 0.0)
                o = (jnp.sum(h1 * w2_r[...], axis=1, keepdims=True)
                     + b2_r[...])
                o_out[...] = jax.nn.sigmoid(o)

        extra_in = [pl.BlockSpec((128, hd), lambda p, i: (0, 0)),
                    pl.BlockSpec((1, hd), lambda p, i: (0, 0)),
                    pl.BlockSpec((1, hd), lambda p, i: (0, 0)),
                    pl.BlockSpec((1, 1), lambda p, i: (0, 0))]
        out_spec = pl.BlockSpec((BN, 1), lambda p, i: (p * i, 0))
        out_shape = jax.ShapeDtypeStruct((n, 1), jnp.float32)
        args = (w1, b1.reshape(1, hd), w2.reshape(1, hd),
                b2.reshape(1, 1))

    return pl.pallas_call(
        kern,
        grid=(2, nb),
        in_specs=[pl.BlockSpec((BN, d), lambda p, i: ((1 - p) * i, 0)),
                  pl.BlockSpec((BN, d), lambda p, i: ((1 - p) * i, 0)),
                  pl.BlockSpec((BN, 1), lambda p, i: ((1 - p) * i, 0)),
                  pl.BlockSpec((d, 128), lambda p, i: (0, 0)),
                  pl.BlockSpec((d, 128), lambda p, i: (0, 0)),
                  pl.BlockSpec((1, 128), lambda p, i: (0, 0)),
                  pl.BlockSpec((1, 128), lambda p, i: (0, 0)),
                  pl.BlockSpec((1, 128), lambda p, i: (0, 0))] + extra_in,
        out_specs=out_spec,
        out_shape=out_shape,
        scratch_shapes=[pltpu.VMEM((n, 128), jnp.float32),
                        pltpu.VMEM((2, 128), jnp.float32)],
    )(agg, h, cnt, wl, wr, bl.reshape(1, 128), g.reshape(1, 128),
      bb.reshape(1, 128), *args)


def kernel(x, edge_index, params):
    n, in_dim = x.shape
    e = edge_index.shape[1]
    src, dst = edge_index[0], edge_index[1]

    unit = GROUP * NS * SUPER
    e_pad = ((e + unit - 1) // unit) * unit
    pad = e_pad - e
    z_sl = ((-(-n // NS) + 127) // 128) * 128
    n_out = z_sl * NS
    n_acc = max(n_out, n + PAD_ROWS)

    ar = jnp.arange(pad, dtype=jnp.int32)
    src2 = jnp.concatenate([src, ar % n]).reshape(-1, GROUP)
    dst2 = jnp.concatenate([dst, n + (ar % PAD_ROWS)]).reshape(-1, GROUP)
    zeros32 = jnp.zeros((z_sl, CHUNK), jnp.float32)
    zeros1 = jnp.zeros((z_sl,), jnp.float32)

    agg2 = _make_agg(2, n_acc, n_out, e_pad, with_count=True)
    agg4 = _make_agg(4, n_acc, n_out, e_pad)

    h = x
    cnt = None
    for i in range(3):
        nch = h.shape[1] // CHUNK
        tbl = h.reshape(n * nch, CHUNK)
        if i == 0:
            agg, cnt_v = agg2(tbl, src2, dst2, zeros32, zeros1)
            cnt = cnt_v.reshape(n_out, 1)
        else:
            (agg,) = agg4(tbl, src2, dst2, zeros32)
        head = None
        if i == 2:
            head = (params['fc1_W'], params['fc1_b'],
                    params['fc2_W'], params['fc2_b'])
        h = _layer_call(agg, h, cnt, params['Wl%d' % i],
                        params['Wr%d' % i], params['bl%d' % i],
                        params['bn_g%d' % i], params['bn_b%d' % i], n,
                        head=head)
    return h


# speedup vs baseline: 1.2147x; 1.2147x over previous
import jax
import jax.numpy as jnp
from jax import lax
from jax.experimental import pallas as pl
from jax.experimental.pallas import tpu as pltpu
from jax.experimental.pallas import tpu_sc as plsc

NC, NS = 2, 16
CHUNK = 32
GROUP = 128
SUPER = 8
PAD_ROWS = 64
EPS = 1e-5
BN = 1000


def _mesh():
    return plsc.VectorSubcoreMesh(core_axis_name="c", subcore_axis_name="s",
                                  num_cores=NC, num_subcores=NS)


def _make_agg(n_chunks, n_acc, n_out, e_pad, with_count=False):
    per_core = n_chunks // NC
    g_total = e_pad // GROUP
    g_tile = g_total // NS
    nsb = g_tile // SUPER
    z_sl = n_out // NS

    def chunk_pass(j, tbl, out, src2, dst2, zeros, acc, si, s4, di, rb,
                   gsem, ssem, isem, s, cnt_refs):
        pltpu.sync_copy(zeros, acc.at[pl.ds(s * z_sl, z_sl)])
        if cnt_refs is not None:
            acc_cnt, ones, zeros1, cnt_out = cnt_refs
            for k in range(GROUP // 16):
                ones[pl.ds(k * 16, 16)] = jnp.ones((16,), jnp.float32)
            pltpu.sync_copy(zeros1, acc_cnt.at[pl.ds(s * z_sl, z_sl)])
        plsc.subcore_barrier()
        g0 = s * g_tile

        def load_idx(sbk, sl):
            g = g0 + sbk * SUPER
            pltpu.async_copy(src2.at[pl.ds(g, SUPER)], si[sl], isem)
            pltpu.async_copy(dst2.at[pl.ds(g, SUPER)], di[sl], isem)

        def wait_idx(sbk, sl):
            g = g0 + sbk * SUPER
            pltpu.make_async_copy(src2.at[pl.ds(g, SUPER)], si[sl],
                                  isem).wait()
            pltpu.make_async_copy(dst2.at[pl.ds(g, SUPER)], di[sl],
                                  isem).wait()

        def calc_s4(sl):
            for gi in range(SUPER):
                for k in range(GROUP // 16):
                    s4[sl][gi, pl.ds(k * 16, 16)] = (
                        si[sl][gi, pl.ds(k * 16, 16)] * n_chunks + j)

        def gissue(sbk, gi, isl, rsl):
            pltpu.async_copy(tbl.at[s4[isl].at[gi]], rb[rsl], gsem[rsl])

        def gwait(isl, gi, rsl):
            pltpu.make_async_copy(tbl.at[s4[isl].at[gi]], rb[rsl],
                                  gsem[rsl]).wait()

        def sissue(isl, gi, rsl):
            pltpu.async_copy(rb[rsl], acc.at[di[isl].at[gi]],
                             ssem[rsl], add=True)
            if cnt_refs is not None:
                pltpu.async_copy(ones, acc_cnt.at[di[isl].at[gi]],
                                 ssem[rsl], add=True)

        def swait(rsl):
            pltpu.make_async_copy(rb[rsl], acc.at[pl.ds(0, GROUP)],
                                  ssem[rsl]).wait()
            if cnt_refs is not None:
                pltpu.make_async_copy(ones, acc_cnt.at[pl.ds(0, GROUP)],
                                      ssem[rsl]).wait()

        load_idx(0, 0)
        wait_idx(0, 0)
        calc_s4(0)
        for gg in range(3):
            gissue(0, gg, 0, gg)

        def sblock(sbk, carry):
            isl_d = lax.rem(sbk, 2)

            def do(cur_par):
                isl = cur_par
                nxt = 1 - cur_par
                for gi in range(SUPER):
                    rsl = gi % 4
                    gwait(isl, gi, rsl)
                    sissue(isl, gi, rsl)
                    nsl = (gi + 3) % 4
                    if gi == 0:
                        pl.when(sbk > 0)(lambda: swait(nsl))
                        pl.when(sbk < nsb - 1)(
                            lambda: load_idx(sbk + 1, nxt))
                    else:
                        swait(nsl)
                    if gi == 4:
                        def prep():
                            wait_idx(sbk + 1, nxt)
                            calc_s4(nxt)
                        pl.when(sbk < nsb - 1)(prep)
                    if gi < 5:
                        gissue(sbk, gi + 3, isl, nsl)
                    else:
                        pl.when(sbk < nsb - 1)(
                            lambda gi=gi, nsl=nsl:
                            gissue(sbk + 1, gi - 5, nxt, nsl))

            for par in range(2):
                pl.when(isl_d == par)(lambda par=par: do(par))
            return carry

        lax.fori_loop(0, nsb, sblock, 0)
        swait((g_tile - 1) % 4)
        plsc.subcore_barrier()
        pltpu.sync_copy(acc.at[pl.ds(s * z_sl, z_sl)],
                        out.at[pl.ds(s * z_sl, z_sl),
                               pl.ds(j * CHUNK, CHUNK)])
        if cnt_refs is not None:
            pltpu.sync_copy(acc_cnt.at[pl.ds(s * z_sl, z_sl)],
                            cnt_out.at[pl.ds(s * z_sl, z_sl)])
        plsc.subcore_barrier()

    n_in = 5 if with_count else 4

    def body(*refs):
        tbl = refs[0]
        src2, dst2, zeros = refs[1:4]
        zeros1 = refs[4] if with_count else None
        out = refs[n_in]
        cnt_out = refs[n_in + 1] if with_count else None
        sc = refs[n_in + (2 if with_count else 1):]
        acc = sc[0]
        si, s4, di = sc[1:3], sc[3:5], sc[5:7]
        rb = sc[7:11]
        gsem, ssem = sc[11:15], sc[15:19]
        isem = sc[19]
        acc_cnt = sc[20] if with_count else None
        ones = sc[21] if with_count else None
        c = lax.axis_index("c")
        s = lax.axis_index("s")
        for cc in range(NC):
            def run(cc=cc):
                for p in range(per_core):
                    j = cc * per_core + p
                    cr = None
                    if with_count and cc == 0 and p == 0:
                        cr = (acc_cnt, ones, zeros1, cnt_out)
                    chunk_pass(j, tbl, out, src2, dst2, zeros,
                               acc, si, s4, di, rb, gsem, ssem, isem, s, cr)
            pl.when(c == cc)(run)

    out_type = (jax.ShapeDtypeStruct((n_out, n_chunks * CHUNK),
                                     jnp.float32),)
    if with_count:
        out_type = out_type + (jax.ShapeDtypeStruct((n_out,), jnp.float32),)
    scratch = (
        [pltpu.VMEM_SHARED((n_acc, CHUNK), jnp.float32)]
        + [pltpu.VMEM((SUPER, GROUP), jnp.int32) for _ in range(6)]
        + [pltpu.VMEM((GROUP, CHUNK), jnp.float32) for _ in range(4)]
        + [pltpu.SemaphoreType.DMA for _ in range(9)]
    )
    if with_count:
        scratch += [
            pltpu.VMEM_SHARED((n_acc,), jnp.float32),
            pltpu.VMEM((GROUP,), jnp.float32),
        ]
    return pl.kernel(body, out_type=out_type, mesh=_mesh(),
                     scratch_types=scratch,
                     compiler_params=pltpu.CompilerParams(
                         use_tc_tiling_on_sc=False))


def _layer_call(agg, h, cnt, wl, wr, bl, g, bb, n, head=None):
    nb = n // BN
    d = wl.shape[0]

    def common_z(agg_r, h_r, cnt_r, wl_r, wr_r, bl_r, z_scr, st_scr, i):
        @pl.when(i == 0)
        def _():
            st_scr[...] = jnp.zeros_like(st_scr)
        inv = 1.0 / jnp.maximum(cnt_r[...][:, 0], 1.0)
        z = (jnp.dot(agg_r[...], wl_r[...],
                     preferred_element_type=jnp.float32) * inv[:, None]
             + jnp.dot(h_r[...], wr_r[...],
                       preferred_element_type=jnp.float32)
             + bl_r[...])
        z_scr[pl.ds(i * BN, BN), :] = z
        st_scr[...] += jnp.stack([jnp.sum(z, axis=0),
                                  jnp.sum(z * z, axis=0)])

    def norm(z_scr, st_scr, g_r, b_r, i):
        stats = st_scr[...]
        mu = stats[0] * (1.0 / n)
        var = stats[1] * (1.0 / n) - mu * mu
        z = z_scr[pl.ds(i * BN, BN), :]
        return jnp.maximum(
            g_r[...] * (z - mu[None, :]) / jnp.sqrt(var + EPS)[None, :]
            + b_r[...], 0.0)

    if head is None:
        def kern(agg_r, h_r, cnt_r, wl_r, wr_r, bl_r, g_r, b_r,
                 h_out, z_scr, st_scr):
            p, i = pl.program_id(0), pl.program_id(1)
            pl.when(p == 0)(lambda: common_z(agg_r, h_r, cnt_r, wl_r,
                                             wr_r, bl_r, z_scr, st_scr, i))

            @pl.when(p == 1)
            def _():
                h_out[...] = norm(z_scr, st_scr, g_r, b_r, i)

        extra_in = []
        out_spec = pl.BlockSpec((BN, 128), lambda p, i: (p * i, 0))
        out_shape = jax.ShapeDtypeStruct((n, 128), jnp.float32)
        args = ()
    else:
        w1, b1, w2, b2 = head
        hd = w1.shape[1]

        def kern(agg_r, h_r, cnt_r, wl_r, wr_r, bl_r, g_r, b_r,
                 w1_r, b1_r, w2_r, b2_r, o_out, z_scr, st_scr):
            p, i = pl.program_id(0), pl.program_id(1)
            pl.when(p == 0)(lambda: common_z(agg_r, h_r, cnt_r, wl_r,
                                             wr_r, bl_r, z_scr, st_scr, i))

            @pl.when(p == 1)
            def _():
                hh = norm(z_scr, st_scr, g_r, b_r, i)
                h1 = jnp.maximum(
                    jnp.dot(hh, w1_r[...],
                            preferred_element_type=jnp.float32)
                    + b1_r[...], 0.0)
                o = (jnp.sum(h1 * w2_r[...], axis=1, keepdims=True)
                     + b2_r[...])
                o_out[...] = jax.nn.sigmoid(o)

        extra_in = [pl.BlockSpec((128, hd), lambda p, i: (0, 0)),
                    pl.BlockSpec((1, hd), lambda p, i: (0, 0)),
                    pl.BlockSpec((1, hd), lambda p, i: (0, 0)),
                    pl.BlockSpec((1, 1), lambda p, i: (0, 0))]
        out_spec = pl.BlockSpec((BN, 1), lambda p, i: (p * i, 0))
        out_shape = jax.ShapeDtypeStruct((n, 1), jnp.float32)
        args = (w1, b1.reshape(1, hd), w2.reshape(1, hd),
                b2.reshape(1, 1))

    return pl.pallas_call(
        kern,
        grid=(2, nb),
        in_specs=[pl.BlockSpec((BN, d), lambda p, i: ((1 - p) * i, 0)),
                  pl.BlockSpec((BN, d), lambda p, i: ((1 - p) * i, 0)),
                  pl.BlockSpec((BN, 1), lambda p, i: ((1 - p) * i, 0)),
                  pl.BlockSpec((d, 128), lambda p, i: (0, 0)),
                  pl.BlockSpec((d, 128), lambda p, i: (0, 0)),
                  pl.BlockSpec((1, 128), lambda p, i: (0, 0)),
                  pl.BlockSpec((1, 128), lambda p, i: (0, 0)),
                  pl.BlockSpec((1, 128), lambda p, i: (0, 0))] + extra_in,
        out_specs=out_spec,
        out_shape=out_shape,
        scratch_shapes=[pltpu.VMEM((n, 128), jnp.float32),
                        pltpu.VMEM((2, 128), jnp.float32)],
    )(agg, h, cnt, wl, wr, bl.reshape(1, 128), g.reshape(1, 128),
      bb.reshape(1, 128), *args)


def kernel(x, edge_index, params):
    n, in_dim = x.shape
    e = edge_index.shape[1]
    src, dst = edge_index[0], edge_index[1]

    unit = GROUP * NS * SUPER
    e_pad = ((e + unit - 1) // unit) * unit
    pad = e_pad - e
    z_sl = ((-(-n // NS) + 127) // 128) * 128
    n_out = z_sl * NS
    n_acc = max(n_out, n + PAD_ROWS)

    ar = jnp.arange(pad, dtype=jnp.int32)
    src2 = jnp.concatenate([src, ar % n]).reshape(-1, GROUP)
    dst2 = jnp.concatenate([dst, n + (ar % PAD_ROWS)]).reshape(-1, GROUP)
    zeros32 = jnp.zeros((z_sl, CHUNK), jnp.float32)
    zeros1 = jnp.zeros((z_sl,), jnp.float32)

    agg2 = _make_agg(2, n_acc, n_out, e_pad, with_count=True)
    agg4 = _make_agg(4, n_acc, n_out, e_pad)

    h = x
    cnt = None
    for i in range(3):
        nch = h.shape[1] // CHUNK
        tbl = h.reshape(n * nch, CHUNK)
        if i == 0:
            agg, cnt_v = agg2(tbl, src2, dst2, zeros32, zeros1)
            cnt = cnt_v.reshape(n_out, 1)
        else:
            (agg,) = agg4(tbl, src2, dst2, zeros32)
        head = None
        if i == 2:
            head = (params['fc1_W'], params['fc1_b'],
                    params['fc2_W'], params['fc2_b'])
        h = _layer_call(agg, h, cnt, params['Wl%d' % i],
                        params['Wr%d' % i], params['bl%d' % i],
                        params['bn_g%d' % i], params['bn_b%d' % i], n,
                        head=head)
    return h


# speedup vs baseline: 1.3032x; 1.0728x over previous
import jax
import jax.numpy as jnp
from jax import lax
from jax.experimental import pallas as pl
from jax.experimental.pallas import tpu as pltpu
from jax.experimental.pallas import tpu_sc as plsc

NC, NS = 2, 16
CHUNK = 32
GROUP = 128
SUPER = 8
PAD_ROWS = 64
EPS = 1e-5
BN = 2000


def _mesh():
    return plsc.VectorSubcoreMesh(core_axis_name="c", subcore_axis_name="s",
                                  num_cores=NC, num_subcores=NS)


def _make_agg(n_chunks, n_acc, n_out, e_pad, with_count=False):
    per_core = n_chunks // NC
    g_total = e_pad // GROUP
    g_tile = g_total // NS
    nsb = g_tile // SUPER
    z_sl = n_out // NS

    def chunk_pass(j, tbl, out, src2, dst2, zeros, acc, si, s4, di, rb,
                   gsem, ssem, isem, s, cnt_refs):
        pltpu.sync_copy(zeros, acc.at[pl.ds(s * z_sl, z_sl)])
        if cnt_refs is not None:
            acc_cnt, ones, zeros1, cnt_out = cnt_refs
            for k in range(GROUP // 16):
                ones[pl.ds(k * 16, 16)] = jnp.ones((16,), jnp.float32)
            pltpu.sync_copy(zeros1, acc_cnt.at[pl.ds(s * z_sl, z_sl)])
        plsc.subcore_barrier()
        g0 = s * g_tile

        def load_idx(sbk, sl):
            g = g0 + sbk * SUPER
            pltpu.async_copy(src2.at[pl.ds(g, SUPER)], si[sl], isem)
            pltpu.async_copy(dst2.at[pl.ds(g, SUPER)], di[sl], isem)

        def wait_idx(sbk, sl):
            g = g0 + sbk * SUPER
            pltpu.make_async_copy(src2.at[pl.ds(g, SUPER)], si[sl],
                                  isem).wait()
            pltpu.make_async_copy(dst2.at[pl.ds(g, SUPER)], di[sl],
                                  isem).wait()

        def calc_s4(sl):
            for gi in range(SUPER):
                for k in range(GROUP // 16):
                    s4[sl][gi, pl.ds(k * 16, 16)] = (
                        si[sl][gi, pl.ds(k * 16, 16)] * n_chunks + j)

        def gissue(sbk, gi, isl, rsl):
            pltpu.async_copy(tbl.at[s4[isl].at[gi]], rb[rsl], gsem[rsl])

        def gwait(isl, gi, rsl):
            pltpu.make_async_copy(tbl.at[s4[isl].at[gi]], rb[rsl],
                                  gsem[rsl]).wait()

        def sissue(isl, gi, rsl):
            pltpu.async_copy(rb[rsl], acc.at[di[isl].at[gi]],
                             ssem[rsl], add=True)
            if cnt_refs is not None:
                pltpu.async_copy(ones, acc_cnt.at[di[isl].at[gi]],
                                 ssem[rsl], add=True)

        def swait(rsl):
            pltpu.make_async_copy(rb[rsl], acc.at[pl.ds(0, GROUP)],
                                  ssem[rsl]).wait()
            if cnt_refs is not None:
                pltpu.make_async_copy(ones, acc_cnt.at[pl.ds(0, GROUP)],
                                      ssem[rsl]).wait()

        load_idx(0, 0)
        wait_idx(0, 0)
        calc_s4(0)
        for gg in range(3):
            gissue(0, gg, 0, gg)

        def sblock(sbk, carry):
            isl_d = lax.rem(sbk, 2)

            def do(cur_par):
                isl = cur_par
                nxt = 1 - cur_par
                for gi in range(SUPER):
                    rsl = gi % 4
                    gwait(isl, gi, rsl)
                    sissue(isl, gi, rsl)
                    nsl = (gi + 3) % 4
                    if gi == 0:
                        pl.when(sbk > 0)(lambda: swait(nsl))
                        pl.when(sbk < nsb - 1)(
                            lambda: load_idx(sbk + 1, nxt))
                    else:
                        swait(nsl)
                    if gi == 4:
                        def prep():
                            wait_idx(sbk + 1, nxt)
                            calc_s4(nxt)
                        pl.when(sbk < nsb - 1)(prep)
                    if gi < 5:
                        gissue(sbk, gi + 3, isl, nsl)
                    else:
                        pl.when(sbk < nsb - 1)(
                            lambda gi=gi, nsl=nsl:
                            gissue(sbk + 1, gi - 5, nxt, nsl))

            for par in range(2):
                pl.when(isl_d == par)(lambda par=par: do(par))
            return carry

        lax.fori_loop(0, nsb, sblock, 0)
        swait((g_tile - 1) % 4)
        plsc.subcore_barrier()
        pltpu.sync_copy(acc.at[pl.ds(s * z_sl, z_sl)],
                        out.at[pl.ds(s * z_sl, z_sl),
                               pl.ds(j * CHUNK, CHUNK)])
        if cnt_refs is not None:
            pltpu.sync_copy(acc_cnt.at[pl.ds(s * z_sl, z_sl)],
                            cnt_out.at[pl.ds(s * z_sl, z_sl)])
        plsc.subcore_barrier()

    n_in = 5 if with_count else 4

    def body(*refs):
        tbl = refs[0]
        src2, dst2, zeros = refs[1:4]
        zeros1 = refs[4] if with_count else None
        out = refs[n_in]
        cnt_out = refs[n_in + 1] if with_count else None
        sc = refs[n_in + (2 if with_count else 1):]
        acc = sc[0]
        si, s4, di = sc[1:3], sc[3:5], sc[5:7]
        rb = sc[7:11]
        gsem, ssem = sc[11:15], sc[15:19]
        isem = sc[19]
        acc_cnt = sc[20] if with_count else None
        ones = sc[21] if with_count else None
        c = lax.axis_index("c")
        s = lax.axis_index("s")
        for cc in range(NC):
            def run(cc=cc):
                for p in range(per_core):
                    j = cc * per_core + p
                    cr = None
                    if with_count and cc == 0 and p == 0:
                        cr = (acc_cnt, ones, zeros1, cnt_out)
                    chunk_pass(j, tbl, out, src2, dst2, zeros,
                               acc, si, s4, di, rb, gsem, ssem, isem, s, cr)
            pl.when(c == cc)(run)

    out_type = (jax.ShapeDtypeStruct((n_out, n_chunks * CHUNK),
                                     jnp.float32),)
    if with_count:
        out_type = out_type + (jax.ShapeDtypeStruct((n_out,), jnp.float32),)
    scratch = (
        [pltpu.VMEM_SHARED((n_acc, CHUNK), jnp.float32)]
        + [pltpu.VMEM((SUPER, GROUP), jnp.int32) for _ in range(6)]
        + [pltpu.VMEM((GROUP, CHUNK), jnp.float32) for _ in range(4)]
        + [pltpu.SemaphoreType.DMA for _ in range(9)]
    )
    if with_count:
        scratch += [
            pltpu.VMEM_SHARED((n_acc,), jnp.float32),
            pltpu.VMEM((GROUP,), jnp.float32),
        ]
    return pl.kernel(body, out_type=out_type, mesh=_mesh(),
                     scratch_types=scratch,
                     compiler_params=pltpu.CompilerParams(
                         use_tc_tiling_on_sc=False))


def _layer_call(agg, h, cnt, wl, wr, bl, g, bb, n, head=None):
    nb = n // BN
    d = wl.shape[0]

    def common_z(agg_r, h_r, cnt_r, wl_r, wr_r, bl_r, z_scr, st_scr, i):
        @pl.when(i == 0)
        def _():
            st_scr[...] = jnp.zeros_like(st_scr)
        inv = 1.0 / jnp.maximum(cnt_r[...][:, 0], 1.0)
        z = (jnp.dot(agg_r[...], wl_r[...],
                     preferred_element_type=jnp.float32) * inv[:, None]
             + jnp.dot(h_r[...], wr_r[...],
                       preferred_element_type=jnp.float32)
             + bl_r[...])
        z_scr[pl.ds(i * BN, BN), :] = z
        st_scr[...] += jnp.stack([jnp.sum(z, axis=0),
                                  jnp.sum(z * z, axis=0)])

    def norm(z_scr, st_scr, g_r, b_r, i):
        stats = st_scr[...]
        mu = stats[0] * (1.0 / n)
        var = stats[1] * (1.0 / n) - mu * mu
        z = z_scr[pl.ds(i * BN, BN), :]
        return jnp.maximum(
            g_r[...] * (z - mu[None, :]) / jnp.sqrt(var + EPS)[None, :]
            + b_r[...], 0.0)

    if head is None:
        def kern(agg_r, h_r, cnt_r, wl_r, wr_r, bl_r, g_r, b_r,
                 h_out, z_scr, st_scr):
            p, i = pl.program_id(0), pl.program_id(1)
            pl.when(p == 0)(lambda: common_z(agg_r, h_r, cnt_r, wl_r,
                                             wr_r, bl_r, z_scr, st_scr, i))

            @pl.when(p == 1)
            def _():
                h_out[...] = norm(z_scr, st_scr, g_r, b_r, i)

        extra_in = []
        out_spec = pl.BlockSpec((BN, 128), lambda p, i: (p * i, 0))
        out_shape = jax.ShapeDtypeStruct((n, 128), jnp.float32)
        args = ()
    else:
        w1, b1, w2, b2 = head
        hd = w1.shape[1]

        def kern(agg_r, h_r, cnt_r, wl_r, wr_r, bl_r, g_r, b_r,
                 w1_r, b1_r, w2_r, b2_r, o_out, z_scr, st_scr):
            p, i = pl.program_id(0), pl.program_id(1)
            pl.when(p == 0)(lambda: common_z(agg_r, h_r, cnt_r, wl_r,
                                             wr_r, bl_r, z_scr, st_scr, i))

            @pl.when(p == 1)
            def _():
                hh = norm(z_scr, st_scr, g_r, b_r, i)
                h1 = jnp.maximum(
                    jnp.dot(hh, w1_r[...],
                            preferred_element_type=jnp.float32)
                    + b1_r[...], 0.0)
                o = (jnp.sum(h1 * w2_r[...], axis=1, keepdims=True)
                     + b2_r[...])
                o_out[...] = jax.nn.sigmoid(o)

        extra_in = [pl.BlockSpec((128, hd), lambda p, i: (0, 0)),
                    pl.BlockSpec((1, hd), lambda p, i: (0, 0)),
                    pl.BlockSpec((1, hd), lambda p, i: (0, 0)),
                    pl.BlockSpec((1, 1), lambda p, i: (0, 0))]
        out_spec = pl.BlockSpec((BN, 1), lambda p, i: (p * i, 0))
        out_shape = jax.ShapeDtypeStruct((n, 1), jnp.float32)
        args = (w1, b1.reshape(1, hd), w2.reshape(1, hd),
                b2.reshape(1, 1))

    return pl.pallas_call(
        kern,
        grid=(2, nb),
        in_specs=[pl.BlockSpec((BN, d), lambda p, i: ((1 - p) * i, 0)),
                  pl.BlockSpec((BN, d), lambda p, i: ((1 - p) * i, 0)),
                  pl.BlockSpec((BN, 1), lambda p, i: ((1 - p) * i, 0)),
                  pl.BlockSpec((d, 128), lambda p, i: (0, 0)),
                  pl.BlockSpec((d, 128), lambda p, i: (0, 0)),
                  pl.BlockSpec((1, 128), lambda p, i: (0, 0)),
                  pl.BlockSpec((1, 128), lambda p, i: (0, 0)),
                  pl.BlockSpec((1, 128), lambda p, i: (0, 0))] + extra_in,
        out_specs=out_spec,
        out_shape=out_shape,
        scratch_shapes=[pltpu.VMEM((n, 128), jnp.float32),
                        pltpu.VMEM((2, 128), jnp.float32)],
    )(agg, h, cnt, wl, wr, bl.reshape(1, 128), g.reshape(1, 128),
      bb.reshape(1, 128), *args)


def kernel(x, edge_index, params):
    n, in_dim = x.shape
    e = edge_index.shape[1]
    src, dst = edge_index[0], edge_index[1]

    unit = GROUP * NS * SUPER
    e_pad = ((e + unit - 1) // unit) * unit
    pad = e_pad - e
    z_sl = ((-(-n // NS) + 127) // 128) * 128
    n_out = z_sl * NS
    n_acc = max(n_out, n + PAD_ROWS)

    ar = jnp.arange(pad, dtype=jnp.int32)
    src2 = jnp.concatenate([src, ar % n]).reshape(-1, GROUP)
    dst2 = jnp.concatenate([dst, n + (ar % PAD_ROWS)]).reshape(-1, GROUP)
    zeros32 = jnp.zeros((z_sl, CHUNK), jnp.float32)
    zeros1 = jnp.zeros((z_sl,), jnp.float32)

    agg2 = _make_agg(2, n_acc, n_out, e_pad, with_count=True)
    agg4 = _make_agg(4, n_acc, n_out, e_pad)

    h = x
    cnt = None
    for i in range(3):
        nch = h.shape[1] // CHUNK
        tbl = h.reshape(n * nch, CHUNK)
        if i == 0:
            agg, cnt_v = agg2(tbl, src2, dst2, zeros32, zeros1)
            cnt = cnt_v.reshape(n_out, 1)
        else:
            (agg,) = agg4(tbl, src2, dst2, zeros32)
        head = None
        if i == 2:
            head = (params['fc1_W'], params['fc1_b'],
                    params['fc2_W'], params['fc2_b'])
        h = _layer_call(agg, h, cnt, params['Wl%d' % i],
                        params['Wr%d' % i], params['bl%d' % i],
                        params['bn_g%d' % i], params['bn_b%d' % i], n,
                        head=head)
    return h


# speedup vs baseline: 1.3397x; 1.0280x over previous
import jax
import jax.numpy as jnp
from jax import lax
from jax.experimental import pallas as pl
from jax.experimental.pallas import tpu as pltpu
from jax.experimental.pallas import tpu_sc as plsc

NC, NS = 2, 16
CHUNK = 32
GROUP = 128
SUPER = 8
PAD_ROWS = 64
EPS = 1e-5
BN = 5000


def _mesh():
    return plsc.VectorSubcoreMesh(core_axis_name="c", subcore_axis_name="s",
                                  num_cores=NC, num_subcores=NS)


def _make_agg(n_chunks, n_acc, n_out, e_pad, with_count=False):
    per_core = n_chunks // NC
    g_total = e_pad // GROUP
    g_tile = g_total // NS
    nsb = g_tile // SUPER
    z_sl = n_out // NS

    def chunk_pass(j, tbl, out, src2, dst2, zeros, acc, si, s4, di, rb,
                   gsem, ssem, isem, s, cnt_refs):
        pltpu.sync_copy(zeros, acc.at[pl.ds(s * z_sl, z_sl)])
        if cnt_refs is not None:
            acc_cnt, ones, zeros1, cnt_out = cnt_refs
            for k in range(GROUP // 16):
                ones[pl.ds(k * 16, 16)] = jnp.ones((16,), jnp.float32)
            pltpu.sync_copy(zeros1, acc_cnt.at[pl.ds(s * z_sl, z_sl)])
        plsc.subcore_barrier()
        g0 = s * g_tile

        def load_idx(sbk, sl):
            g = g0 + sbk * SUPER
            pltpu.async_copy(src2.at[pl.ds(g, SUPER)], si[sl], isem)
            pltpu.async_copy(dst2.at[pl.ds(g, SUPER)], di[sl], isem)

        def wait_idx(sbk, sl):
            g = g0 + sbk * SUPER
            pltpu.make_async_copy(src2.at[pl.ds(g, SUPER)], si[sl],
                                  isem).wait()
            pltpu.make_async_copy(dst2.at[pl.ds(g, SUPER)], di[sl],
                                  isem).wait()

        def calc_s4(sl):
            for gi in range(SUPER):
                for k in range(GROUP // 16):
                    s4[sl][gi, pl.ds(k * 16, 16)] = (
                        si[sl][gi, pl.ds(k * 16, 16)] * n_chunks + j)

        def gissue(sbk, gi, isl, rsl):
            pltpu.async_copy(tbl.at[s4[isl].at[gi]], rb[rsl], gsem[rsl])

        def gwait(isl, gi, rsl):
            pltpu.make_async_copy(tbl.at[s4[isl].at[gi]], rb[rsl],
                                  gsem[rsl]).wait()

        def sissue(isl, gi, rsl):
            pltpu.async_copy(rb[rsl], acc.at[di[isl].at[gi]],
                             ssem[rsl], add=True)
            if cnt_refs is not None:
                pltpu.async_copy(ones, acc_cnt.at[di[isl].at[gi]],
                                 ssem[rsl], add=True)

        def swait(rsl):
            pltpu.make_async_copy(rb[rsl], acc.at[pl.ds(0, GROUP)],
                                  ssem[rsl]).wait()
            if cnt_refs is not None:
                pltpu.make_async_copy(ones, acc_cnt.at[pl.ds(0, GROUP)],
                                      ssem[rsl]).wait()

        load_idx(0, 0)
        wait_idx(0, 0)
        calc_s4(0)
        for gg in range(3):
            gissue(0, gg, 0, gg)

        def sblock(sbk, carry):
            isl_d = lax.rem(sbk, 2)

            def do(cur_par):
                isl = cur_par
                nxt = 1 - cur_par
                for gi in range(SUPER):
                    rsl = gi % 4
                    gwait(isl, gi, rsl)
                    sissue(isl, gi, rsl)
                    nsl = (gi + 3) % 4
                    if gi == 0:
                        pl.when(sbk > 0)(lambda: swait(nsl))
                        pl.when(sbk < nsb - 1)(
                            lambda: load_idx(sbk + 1, nxt))
                    else:
                        swait(nsl)
                    if gi == 4:
                        def prep():
                            wait_idx(sbk + 1, nxt)
                            calc_s4(nxt)
                        pl.when(sbk < nsb - 1)(prep)
                    if gi < 5:
                        gissue(sbk, gi + 3, isl, nsl)
                    else:
                        pl.when(sbk < nsb - 1)(
                            lambda gi=gi, nsl=nsl:
                            gissue(sbk + 1, gi - 5, nxt, nsl))

            for par in range(2):
                pl.when(isl_d == par)(lambda par=par: do(par))
            return carry

        lax.fori_loop(0, nsb, sblock, 0)
        swait((g_tile - 1) % 4)
        plsc.subcore_barrier()
        pltpu.sync_copy(acc.at[pl.ds(s * z_sl, z_sl)],
                        out.at[pl.ds(s * z_sl, z_sl),
                               pl.ds(j * CHUNK, CHUNK)])
        if cnt_refs is not None:
            pltpu.sync_copy(acc_cnt.at[pl.ds(s * z_sl, z_sl)],
                            cnt_out.at[pl.ds(s * z_sl, z_sl)])
        plsc.subcore_barrier()

    n_in = 5 if with_count else 4

    def body(*refs):
        tbl = refs[0]
        src2, dst2, zeros = refs[1:4]
        zeros1 = refs[4] if with_count else None
        out = refs[n_in]
        cnt_out = refs[n_in + 1] if with_count else None
        sc = refs[n_in + (2 if with_count else 1):]
        acc = sc[0]
        si, s4, di = sc[1:3], sc[3:5], sc[5:7]
        rb = sc[7:11]
        gsem, ssem = sc[11:15], sc[15:19]
        isem = sc[19]
        acc_cnt = sc[20] if with_count else None
        ones = sc[21] if with_count else None
        c = lax.axis_index("c")
        s = lax.axis_index("s")
        for cc in range(NC):
            def run(cc=cc):
                for p in range(per_core):
                    j = cc * per_core + p
                    cr = None
                    if with_count and cc == 0 and p == 0:
                        cr = (acc_cnt, ones, zeros1, cnt_out)
                    chunk_pass(j, tbl, out, src2, dst2, zeros,
                               acc, si, s4, di, rb, gsem, ssem, isem, s, cr)
            pl.when(c == cc)(run)

    out_type = (jax.ShapeDtypeStruct((n_out, n_chunks * CHUNK),
                                     jnp.float32),)
    if with_count:
        out_type = out_type + (jax.ShapeDtypeStruct((n_out,), jnp.float32),)
    scratch = (
        [pltpu.VMEM_SHARED((n_acc, CHUNK), jnp.float32)]
        + [pltpu.VMEM((SUPER, GROUP), jnp.int32) for _ in range(6)]
        + [pltpu.VMEM((GROUP, CHUNK), jnp.float32) for _ in range(4)]
        + [pltpu.SemaphoreType.DMA for _ in range(9)]
    )
    if with_count:
        scratch += [
            pltpu.VMEM_SHARED((n_acc,), jnp.float32),
            pltpu.VMEM((GROUP,), jnp.float32),
        ]
    return pl.kernel(body, out_type=out_type, mesh=_mesh(),
                     scratch_types=scratch,
                     compiler_params=pltpu.CompilerParams(
                         use_tc_tiling_on_sc=False))


def _layer_call(agg, h, cnt, wl, wr, bl, g, bb, n, head=None):
    nb = n // BN
    d = wl.shape[0]

    def common_z(agg_r, h_r, cnt_r, wl_r, wr_r, bl_r, z_scr, st_scr, i):
        @pl.when(i == 0)
        def _():
            st_scr[...] = jnp.zeros_like(st_scr)
        inv = 1.0 / jnp.maximum(cnt_r[...][:, 0], 1.0)
        z = (jnp.dot(agg_r[...], wl_r[...],
                     preferred_element_type=jnp.float32) * inv[:, None]
             + jnp.dot(h_r[...], wr_r[...],
                       preferred_element_type=jnp.float32)
             + bl_r[...])
        z_scr[pl.ds(i * BN, BN), :] = z
        st_scr[...] += jnp.stack([jnp.sum(z, axis=0),
                                  jnp.sum(z * z, axis=0)])

    def norm(z_scr, st_scr, g_r, b_r, i):
        stats = st_scr[...]
        mu = stats[0] * (1.0 / n)
        var = stats[1] * (1.0 / n) - mu * mu
        z = z_scr[pl.ds(i * BN, BN), :]
        return jnp.maximum(
            g_r[...] * (z - mu[None, :]) / jnp.sqrt(var + EPS)[None, :]
            + b_r[...], 0.0)

    if head is None:
        def kern(agg_r, h_r, cnt_r, wl_r, wr_r, bl_r, g_r, b_r,
                 h_out, z_scr, st_scr):
            p, i = pl.program_id(0), pl.program_id(1)
            pl.when(p == 0)(lambda: common_z(agg_r, h_r, cnt_r, wl_r,
                                             wr_r, bl_r, z_scr, st_scr, i))

            @pl.when(p == 1)
            def _():
                h_out[...] = norm(z_scr, st_scr, g_r, b_r, i)

        extra_in = []
        out_spec = pl.BlockSpec((BN, 128), lambda p, i: (p * i, 0))
        out_shape = jax.ShapeDtypeStruct((n, 128), jnp.float32)
        args = ()
    else:
        w1, b1, w2, b2 = head
        hd = w1.shape[1]

        def kern(agg_r, h_r, cnt_r, wl_r, wr_r, bl_r, g_r, b_r,
                 w1_r, b1_r, w2_r, b2_r, o_out, z_scr, st_scr):
            p, i = pl.program_id(0), pl.program_id(1)
            pl.when(p == 0)(lambda: common_z(agg_r, h_r, cnt_r, wl_r,
                                             wr_r, bl_r, z_scr, st_scr, i))

            @pl.when(p == 1)
            def _():
                hh = norm(z_scr, st_scr, g_r, b_r, i)
                h1 = jnp.maximum(
                    jnp.dot(hh, w1_r[...],
                            preferred_element_type=jnp.float32)
                    + b1_r[...], 0.0)
                o = (jnp.sum(h1 * w2_r[...], axis=1, keepdims=True)
                     + b2_r[...])
                o_out[...] = jax.nn.sigmoid(o)

        extra_in = [pl.BlockSpec((128, hd), lambda p, i: (0, 0)),
                    pl.BlockSpec((1, hd), lambda p, i: (0, 0)),
                    pl.BlockSpec((1, hd), lambda p, i: (0, 0)),
                    pl.BlockSpec((1, 1), lambda p, i: (0, 0))]
        out_spec = pl.BlockSpec((BN, 1), lambda p, i: (p * i, 0))
        out_shape = jax.ShapeDtypeStruct((n, 1), jnp.float32)
        args = (w1, b1.reshape(1, hd), w2.reshape(1, hd),
                b2.reshape(1, 1))

    return pl.pallas_call(
        kern,
        grid=(2, nb),
        in_specs=[pl.BlockSpec((BN, d), lambda p, i: ((1 - p) * i, 0)),
                  pl.BlockSpec((BN, d), lambda p, i: ((1 - p) * i, 0)),
                  pl.BlockSpec((BN, 1), lambda p, i: ((1 - p) * i, 0)),
                  pl.BlockSpec((d, 128), lambda p, i: (0, 0)),
                  pl.BlockSpec((d, 128), lambda p, i: (0, 0)),
                  pl.BlockSpec((1, 128), lambda p, i: (0, 0)),
                  pl.BlockSpec((1, 128), lambda p, i: (0, 0)),
                  pl.BlockSpec((1, 128), lambda p, i: (0, 0))] + extra_in,
        out_specs=out_spec,
        out_shape=out_shape,
        scratch_shapes=[pltpu.VMEM((n, 128), jnp.float32),
                        pltpu.VMEM((2, 128), jnp.float32)],
    )(agg, h, cnt, wl, wr, bl.reshape(1, 128), g.reshape(1, 128),
      bb.reshape(1, 128), *args)


def kernel(x, edge_index, params):
    n, in_dim = x.shape
    e = edge_index.shape[1]
    src, dst = edge_index[0], edge_index[1]

    unit = GROUP * NS * SUPER
    e_pad = ((e + unit - 1) // unit) * unit
    pad = e_pad - e
    z_sl = ((-(-n // NS) + 127) // 128) * 128
    n_out = z_sl * NS
    n_acc = max(n_out, n + PAD_ROWS)

    ar = jnp.arange(pad, dtype=jnp.int32)
    src2 = jnp.concatenate([src, ar % n]).reshape(-1, GROUP)
    dst2 = jnp.concatenate([dst, n + (ar % PAD_ROWS)]).reshape(-1, GROUP)
    zeros32 = jnp.zeros((z_sl, CHUNK), jnp.float32)
    zeros1 = jnp.zeros((z_sl,), jnp.float32)

    agg2 = _make_agg(2, n_acc, n_out, e_pad, with_count=True)
    agg4 = _make_agg(4, n_acc, n_out, e_pad)

    h = x
    cnt = None
    for i in range(3):
        nch = h.shape[1] // CHUNK
        tbl = h.reshape(n * nch, CHUNK)
        if i == 0:
            agg, cnt_v = agg2(tbl, src2, dst2, zeros32, zeros1)
            cnt = cnt_v.reshape(n_out, 1)
        else:
            (agg,) = agg4(tbl, src2, dst2, zeros32)
        head = None
        if i == 2:
            head = (params['fc1_W'], params['fc1_b'],
                    params['fc2_W'], params['fc2_b'])
        h = _layer_call(agg, h, cnt, params['Wl%d' % i],
                        params['Wr%d' % i], params['bl%d' % i],
                        params['bn_g%d' % i], params['bn_b%d' % i], n,
                        head=head)
    return h
